# baseline (speedup 1.0000x reference)
import functools

import jax
import jax.numpy as jnp
from jax import lax
from jax.experimental import pallas as pl
from jax.experimental.pallas import tpu as pltpu
import jax.experimental.pallas.tpu_sc as plsc

_NC, _NS, _L = 2, 16, 16
_NW = _NC * _NS
_NB = 5
_P = 2


def _sc_body(B, S, D, CS, x_hbm, pe_hbm, out_hbm, *refs):
    xb = refs[0:_NB]
    peb = refs[_NB:_NB + 2]
    sin = refs[_NB + 2:2 * _NB + 2]
    sout = refs[2 * _NB + 2:3 * _NB + 2]
    spe = refs[3 * _NB + 2:3 * _NB + 4]
    sh = refs[3 * _NB + 4]

    sid = lax.axis_index("s")
    wid = lax.axis_index("s") * _NC + lax.axis_index("c")
    rows_per_w = S // _NW
    n_chunks = rows_per_w // CS
    s0 = wid * rows_per_w
    gpr = D // _L

    items = [(c, b) for c in range(n_chunks) for b in range(B)]
    n = len(items)

    def rows(i):
        return pl.ds(s0 + items[i][0] * CS, CS)

    def pe_rows(c):
        return pl.ds(s0 + c * CS, CS)

    pe_pend = [None, None]
    for c in range(min(2, n_chunks)):
        pe_pend[c] = pltpu.async_copy(pe_hbm.at[pe_rows(c)], peb[c], spe[c])
    loads = {}
    stores = {}
    for i in range(min(_P, n)):
        loads[i] = pltpu.async_copy(
            x_hbm.at[items[i][1], rows(i)], xb[i % _NB], sin[i % _NB])

    for i in range(n):
        k = i % _NB
        c, b = items[i]
        ni = i + _P
        if ni < n:
            nk = ni % _NB
            loads[ni] = pltpu.async_copy(
                x_hbm.at[items[ni][1], rows(ni)], xb[nk], sin[nk])
        loads[i].wait()
        pk = c % 2
        if pe_pend[pk] is not None:
            pe_pend[pk].wait()
            pe_pend[pk] = None

        xk, pek = xb[k], peb[pk]

        @plsc.parallel_loop(0, CS * gpr, step=1, unroll=8)
        def add_body(g):
            r = g // gpr
            sl = pl.ds((g % gpr) * _L, _L)
            xk[r, sl] = xk[r, sl] + pek[r, sl]

        if i - 1 >= 0 and (i - 1) in stores:
            stores[i - 1].wait()
        pltpu.sync_copy(xk, sh.at[sid, 0])
        stores[i] = pltpu.async_copy(sh.at[sid, 0], out_hbm.at[b, rows(i)], sout[k])
        if b == B - 1 and c + 2 < n_chunks:
            pe_pend[pk] = pltpu.async_copy(
                pe_hbm.at[pe_rows(c + 2)], peb[pk], spe[pk])

    stores[n - 1].wait()


def kernel(x, pe):
    B, S, D = x.shape
    CS = 16
    mesh = plsc.VectorSubcoreMesh(core_axis_name="c", subcore_axis_name="s")
    body = functools.partial(_sc_body, B, S, D, CS)
    scratch = (
        [pltpu.VMEM((CS, D), jnp.float32) for _ in range(_NB + 2)]
        + [pltpu.SemaphoreType.DMA for _ in range(2 * _NB + 2)]
        + [pltpu.VMEM_SHARED((_NS, 1, CS, D), jnp.float32)]
    )
    return pl.kernel(
        body,
        out_type=jax.ShapeDtypeStruct((B, S, D), x.dtype),
        mesh=mesh,
        scratch_types=scratch,
    )(x, pe)


# speedup vs baseline: 1.1527x; 1.1527x over previous
import functools

import jax
import jax.numpy as jnp
from jax import lax
from jax.experimental import pallas as pl
from jax.experimental.pallas import tpu as pltpu
import jax.experimental.pallas.tpu_sc as plsc

_NC, _NS, _L = 2, 16, 16
_NW = _NC * _NS
_NB = 5
_P = 3


def _sc_body(B, S, D, CS, x_hbm, pe_hbm, out_hbm, *refs):
    xb = refs[0:_NB]
    peb = refs[_NB:_NB + 2]
    sin = refs[_NB + 2:2 * _NB + 2]
    sout = refs[2 * _NB + 2:3 * _NB + 2]
    spe = refs[3 * _NB + 2:3 * _NB + 4]

    wid = lax.axis_index("s") * _NC + lax.axis_index("c")
    rows_per_w = S // _NW
    n_chunks = rows_per_w // CS
    s0 = wid * rows_per_w
    gpr = D // _L

    items = [(c, b) for c in range(n_chunks) for b in range(B)]
    n = len(items)

    def rows(i):
        return pl.ds(s0 + items[i][0] * CS, CS)

    def pe_rows(c):
        return pl.ds(s0 + c * CS, CS)

    pe_pend = [None, None]
    for c in range(min(2, n_chunks)):
        pe_pend[c] = pltpu.async_copy(pe_hbm.at[pe_rows(c)], peb[c], spe[c])
    loads = {}
    stores = {}
    for i in range(min(_P, n)):
        loads[i] = pltpu.async_copy(
            x_hbm.at[items[i][1], rows(i)], xb[i % _NB], sin[i % _NB])

    for i in range(n):
        k = i % _NB
        c, b = items[i]
        ni = i + _P
        if ni < n:
            nk = ni % _NB
            if ni - _NB >= 0:
                stores[ni - _NB].wait()
            loads[ni] = pltpu.async_copy(
                x_hbm.at[items[ni][1], rows(ni)], xb[nk], sin[nk])
        loads[i].wait()
        pk = c % 2
        if pe_pend[pk] is not None:
            pe_pend[pk].wait()
            pe_pend[pk] = None

        xk, pek = xb[k], peb[pk]

        @plsc.parallel_loop(0, CS * gpr, step=1, unroll=8)
        def add_body(g):
            r = g // gpr
            sl = pl.ds((g % gpr) * _L, _L)
            xk[r, sl] = xk[r, sl] + pek[r, sl]

        stores[i] = pltpu.async_copy(xk, out_hbm.at[b, rows(i)], sout[k])
        if b == B - 1 and c + 2 < n_chunks:
            pe_pend[pk] = pltpu.async_copy(
                pe_hbm.at[pe_rows(c + 2)], peb[pk], spe[pk])

    for i in range(max(0, n - _NB), n):
        stores[i].wait()


def kernel(x, pe):
    B, S, D = x.shape
    CS = 16
    mesh = plsc.VectorSubcoreMesh(core_axis_name="c", subcore_axis_name="s")
    body = functools.partial(_sc_body, B, S, D, CS)
    scratch = (
        [pltpu.VMEM((CS, D), jnp.float32) for _ in range(_NB + 2)]
        + [pltpu.SemaphoreType.DMA for _ in range(2 * _NB + 2)]
    )
    return pl.kernel(
        body,
        out_type=jax.ShapeDtypeStruct((B, S, D), x.dtype),
        mesh=mesh,
        scratch_types=scratch,
    )(x, pe)


# speedup vs baseline: 1.7404x; 1.5099x over previous
import jax
import jax.numpy as jnp
from jax.experimental import pallas as pl


def _add_body(x_ref, pe_ref, o_ref):
    o_ref[...] = x_ref[...] + pe_ref[...]


def kernel(x, pe):
    B, S, D = x.shape
    SB = 2048
    DB = 512
    grid = (S // SB, D // DB, B)
    return pl.pallas_call(
        _add_body,
        grid=grid,
        in_specs=[
            pl.BlockSpec((1, SB, DB), lambda j, d, b: (b, j, d)),
            pl.BlockSpec((SB, DB), lambda j, d, b: (j, d)),
        ],
        out_specs=pl.BlockSpec((1, SB, DB), lambda j, d, b: (b, j, d)),
        out_shape=jax.ShapeDtypeStruct((B, S, D), x.dtype),
    )(x, pe)
